# baseline (speedup 1.0000x reference)
import functools

import jax
import jax.numpy as jnp
from jax import lax
from jax.experimental import pallas as pl
from jax.experimental.pallas import tpu as pltpu
from jax.experimental.pallas import tpu_sc as plsc

BATCH = 16384
LANES = 16


def _make_kernel(num_cores, num_subcores):
    nw = num_cores * num_subcores
    pairs_per_w = BATCH // nw
    ids_per_w = 2 * pairs_per_w
    streams = ids_per_w // 128
    groups = pairs_per_w // LANES

    mesh = plsc.VectorSubcoreMesh(core_axis_name="c", subcore_axis_name="s")

    @functools.partial(
        pl.kernel,
        mesh=mesh,
        out_type=jax.ShapeDtypeStruct((BATCH,), jnp.float32),
        scratch_types=[
            pltpu.VMEM((ids_per_w,), jnp.int32),
            pltpu.VMEM((ids_per_w,), jnp.float32),
            pltpu.VMEM((pairs_per_w,), jnp.float32),
            pltpu.SemaphoreType.DMA,
        ],
        compiler_params=pltpu.CompilerParams(
            needs_layout_passes=False,
            disable_bounds_checks=True,
            disable_semaphore_checks=True,
        ),
    )
    def bt_kernel(xf_hbm, s_hbm, out_hbm, idx_v, val_v, out_v, sem):
        wid = lax.axis_index("s") * num_cores + lax.axis_index("c")
        pltpu.sync_copy(xf_hbm.at[pl.ds(wid * ids_per_w, ids_per_w)], idx_v)
        copies = [
            pltpu.async_copy(
                s_hbm.at[idx_v.at[pl.ds(j * 128, 128)]],
                val_v.at[pl.ds(j * 128, 128)],
                sem,
            )
            for j in range(streams)
        ]
        for c in copies:
            c.wait()
        lanes2 = 2 * lax.iota(jnp.int32, LANES)
        for i in range(groups):
            pos = 2 * LANES * i + lanes2
            s_w = plsc.load_gather(val_v, [pos])
            s_l = plsc.load_gather(val_v, [pos + 1])
            out_v[pl.ds(i * LANES, LANES)] = s_w / (s_w + s_l)
        pltpu.sync_copy(out_v, out_hbm.at[pl.ds(wid * pairs_per_w, pairs_per_w)])

    return bt_kernel


def kernel(x, strengths):
    info = plsc.get_sparse_core_info()
    fn = _make_kernel(info.num_cores, info.num_subcores)
    return fn(x.astype(jnp.int32).reshape(2 * BATCH), strengths)


# speedup vs baseline: 1.1442x; 1.1442x over previous
import functools

import jax
import jax.numpy as jnp
from jax import lax
from jax.experimental import pallas as pl
from jax.experimental.pallas import tpu as pltpu
from jax.experimental.pallas import tpu_sc as plsc

BATCH = 16384
LANES = 16


def _make_kernel(num_cores, num_subcores):
    nw = num_cores * num_subcores
    pairs_per_w = BATCH // nw
    streams = pairs_per_w // 128
    groups = pairs_per_w // LANES

    mesh = plsc.VectorSubcoreMesh(core_axis_name="c", subcore_axis_name="s")

    @functools.partial(
        pl.kernel,
        mesh=mesh,
        out_type=jax.ShapeDtypeStruct((BATCH,), jnp.float32),
        scratch_types=[
            pltpu.VMEM((pairs_per_w, 2), jnp.int32),
            pltpu.VMEM((pairs_per_w,), jnp.int32),
            pltpu.VMEM((pairs_per_w,), jnp.int32),
            pltpu.VMEM((pairs_per_w,), jnp.float32),
            pltpu.VMEM((pairs_per_w,), jnp.float32),
            pltpu.VMEM((pairs_per_w,), jnp.float32),
            pltpu.SemaphoreType.DMA,
        ],
        compiler_params=pltpu.CompilerParams(
            needs_layout_passes=False,
            disable_bounds_checks=True,
            disable_semaphore_checks=True,
        ),
    )
    def bt_kernel(x_hbm, s_hbm, out_hbm, xv, wi_v, li_v, wv, lv, out_v, sem):
        wid = lax.axis_index("s") * num_cores + lax.axis_index("c")
        pltpu.sync_copy(x_hbm.at[pl.ds(wid * pairs_per_w, pairs_per_w), :], xv)
        lanes = lax.iota(jnp.int32, LANES)
        col0 = jnp.zeros((LANES,), jnp.int32)
        col1 = col0 + 1
        for i in range(groups):
            rows = LANES * i + lanes
            wi_v[pl.ds(i * LANES, LANES)] = plsc.load_gather(xv, [rows, col0])
            li_v[pl.ds(i * LANES, LANES)] = plsc.load_gather(xv, [rows, col1])
        copies = [
            pltpu.async_copy(
                s_hbm.at[idx.at[pl.ds(j * 128, 128)]],
                dst.at[pl.ds(j * 128, 128)],
                sem,
            )
            for idx, dst in ((wi_v, wv), (li_v, lv))
            for j in range(streams)
        ]
        for c in copies:
            c.wait()
        for i in range(groups):
            sl = pl.ds(i * LANES, LANES)
            s_w = wv[sl]
            out_v[sl] = s_w / (s_w + lv[sl])
        pltpu.sync_copy(out_v, out_hbm.at[pl.ds(wid * pairs_per_w, pairs_per_w)])

    return bt_kernel


def kernel(x, strengths):
    info = plsc.get_sparse_core_info()
    fn = _make_kernel(info.num_cores, info.num_subcores)
    return fn(x.astype(jnp.int32), strengths)


# speedup vs baseline: 1.1472x; 1.0026x over previous
import functools

import jax
import jax.numpy as jnp
from jax import lax
from jax.experimental import pallas as pl
from jax.experimental.pallas import tpu as pltpu
from jax.experimental.pallas import tpu_sc as plsc

BATCH = 16384
LANES = 16


def _make_kernel(num_cores, num_subcores):
    nw = num_cores * num_subcores
    pairs_per_w = BATCH // nw
    streams = pairs_per_w // 128
    groups = pairs_per_w // LANES

    mesh = plsc.VectorSubcoreMesh(core_axis_name="c", subcore_axis_name="s")

    @functools.partial(
        pl.kernel,
        mesh=mesh,
        out_type=jax.ShapeDtypeStruct((BATCH,), jnp.float32),
        scratch_types=[
            pltpu.VMEM((pairs_per_w, 2), jnp.int32),
            pltpu.VMEM((pairs_per_w,), jnp.int32),
            pltpu.VMEM((pairs_per_w,), jnp.int32),
            pltpu.VMEM((pairs_per_w,), jnp.float32),
            pltpu.VMEM((pairs_per_w,), jnp.float32),
            pltpu.VMEM((pairs_per_w,), jnp.float32),
            pltpu.SemaphoreType.DMA,
        ],
        compiler_params=pltpu.CompilerParams(
            needs_layout_passes=False,
            disable_bounds_checks=True,
            disable_semaphore_checks=True,
            use_tc_tiling_on_sc=True,
        ),
    )
    def bt_kernel(x_hbm, s_hbm, out_hbm, xv, wi_v, li_v, wv, lv, out_v, sem):
        wid = lax.axis_index("s") * num_cores + lax.axis_index("c")
        pltpu.sync_copy(x_hbm.at[pl.ds(wid * pairs_per_w, pairs_per_w), :], xv)
        lanes = lax.iota(jnp.int32, LANES)
        col0 = jnp.zeros((LANES,), jnp.int32)
        col1 = col0 + 1
        for i in range(groups):
            rows = LANES * i + lanes
            wi_v[pl.ds(i * LANES, LANES)] = plsc.load_gather(xv, [rows, col0])
            li_v[pl.ds(i * LANES, LANES)] = plsc.load_gather(xv, [rows, col1])
        copies = [
            pltpu.async_copy(
                s_hbm.at[idx.at[pl.ds(j * 128, 128)]],
                dst.at[pl.ds(j * 128, 128)],
                sem,
            )
            for idx, dst in ((wi_v, wv), (li_v, lv))
            for j in range(streams)
        ]
        for c in copies:
            c.wait()
        for i in range(groups):
            sl = pl.ds(i * LANES, LANES)
            s_w = wv[sl]
            out_v[sl] = s_w / (s_w + lv[sl])
        pltpu.sync_copy(out_v, out_hbm.at[pl.ds(wid * pairs_per_w, pairs_per_w)])

    return bt_kernel


def kernel(x, strengths):
    info = plsc.get_sparse_core_info()
    fn = _make_kernel(info.num_cores, info.num_subcores)
    return fn(x.astype(jnp.int32), strengths)


# speedup vs baseline: 1.4933x; 1.3017x over previous
import functools

import jax
import jax.numpy as jnp
from jax import lax
from jax.experimental import pallas as pl
from jax.experimental.pallas import tpu as pltpu
from jax.experimental.pallas import tpu_sc as plsc

BATCH = 16384
LANES = 16


def _make_kernel(num_cores, num_subcores):
    nw = num_cores * num_subcores
    pairs_per_w = BATCH // nw
    streams = pairs_per_w // 128
    groups = pairs_per_w // LANES

    mesh = plsc.VectorSubcoreMesh(core_axis_name="c", subcore_axis_name="s")

    @functools.partial(
        pl.kernel,
        mesh=mesh,
        out_type=jax.ShapeDtypeStruct((BATCH,), jnp.float32),
        scratch_types=[
            pltpu.VMEM((pairs_per_w,), jnp.int32),
            pltpu.VMEM((pairs_per_w,), jnp.int32),
            pltpu.VMEM((pairs_per_w,), jnp.float32),
            pltpu.VMEM((pairs_per_w,), jnp.float32),
            pltpu.VMEM((pairs_per_w,), jnp.float32),
            pltpu.SemaphoreType.DMA,
        ],
        compiler_params=pltpu.CompilerParams(
            needs_layout_passes=False,
            disable_bounds_checks=True,
            disable_semaphore_checks=True,
        ),
    )
    def bt_kernel(xw_hbm, xl_hbm, s_hbm, out_hbm, wi_v, li_v, wv, lv, out_v, sem):
        wid = lax.axis_index("s") * num_cores + lax.axis_index("c")
        base = wid * pairs_per_w
        stage = [
            pltpu.async_copy(xw_hbm.at[pl.ds(base, pairs_per_w)], wi_v, sem),
            pltpu.async_copy(xl_hbm.at[pl.ds(base, pairs_per_w)], li_v, sem),
        ]
        for c in stage:
            c.wait()
        copies = [
            pltpu.async_copy(
                s_hbm.at[idx.at[pl.ds(j * 128, 128)]],
                dst.at[pl.ds(j * 128, 128)],
                sem,
            )
            for idx, dst in ((wi_v, wv), (li_v, lv))
            for j in range(streams)
        ]
        for c in copies:
            c.wait()
        for i in range(groups):
            sl = pl.ds(i * LANES, LANES)
            s_w = wv[sl]
            out_v[sl] = s_w / (s_w + lv[sl])
        pltpu.sync_copy(out_v, out_hbm.at[pl.ds(base, pairs_per_w)])

    return bt_kernel


def kernel(x, strengths):
    info = plsc.get_sparse_core_info()
    fn = _make_kernel(info.num_cores, info.num_subcores)
    xi = x.astype(jnp.int32)
    return fn(xi[:, 0], xi[:, 1], strengths)


# speedup vs baseline: 1.4972x; 1.0026x over previous
import functools

import jax
import jax.numpy as jnp
from jax import lax
from jax.experimental import pallas as pl
from jax.experimental.pallas import tpu as pltpu
from jax.experimental.pallas import tpu_sc as plsc

BATCH = 16384
LANES = 16


def _make_kernel(num_cores, num_subcores):
    nw = num_cores * num_subcores
    pairs_per_w = BATCH // nw
    streams = pairs_per_w // 128
    groups = pairs_per_w // LANES

    mesh = plsc.VectorSubcoreMesh(core_axis_name="c", subcore_axis_name="s")

    @functools.partial(
        pl.kernel,
        mesh=mesh,
        out_type=jax.ShapeDtypeStruct((BATCH,), jnp.float32),
        scratch_types=[
            pltpu.VMEM((pairs_per_w,), jnp.int32),
            pltpu.VMEM((pairs_per_w,), jnp.int32),
            pltpu.VMEM((pairs_per_w,), jnp.float32),
            pltpu.VMEM((pairs_per_w,), jnp.float32),
            pltpu.VMEM((pairs_per_w,), jnp.float32),
            pltpu.SemaphoreType.DMA,
            pltpu.SemaphoreType.DMA((2, 4)),
            pltpu.SemaphoreType.DMA,
        ],
        compiler_params=pltpu.CompilerParams(
            needs_layout_passes=False,
            disable_bounds_checks=True,
            disable_semaphore_checks=True,
        ),
    )
    def bt_kernel(
        xw_hbm, xl_hbm, s_hbm, out_hbm,
        wi_v, li_v, wv, lv, out_v, sem, gsem, osem,
    ):
        wid = lax.axis_index("s") * num_cores + lax.axis_index("c")
        base = wid * pairs_per_w
        stage = [
            pltpu.async_copy(xw_hbm.at[pl.ds(base, pairs_per_w)], wi_v, sem),
            pltpu.async_copy(xl_hbm.at[pl.ds(base, pairs_per_w)], li_v, sem),
        ]
        for c in stage:
            c.wait()
        copies = [
            [
                pltpu.async_copy(
                    s_hbm.at[idx.at[pl.ds(j * 128, 128)]],
                    dst.at[pl.ds(j * 128, 128)],
                    gsem.at[side, j],
                )
                for j in range(streams)
            ]
            for side, (idx, dst) in enumerate(((wi_v, wv), (li_v, lv)))
        ]
        out_copies = []
        for j in range(streams):
            copies[0][j].wait()
            copies[1][j].wait()
            for i in range(8 * j, 8 * (j + 1)):
                sl = pl.ds(i * LANES, LANES)
                s_w = wv[sl]
                out_v[sl] = s_w / (s_w + lv[sl])
            out_copies.append(
                pltpu.async_copy(
                    out_v.at[pl.ds(j * 128, 128)],
                    out_hbm.at[pl.ds(base + j * 128, 128)],
                    osem,
                )
            )
        for c in out_copies:
            c.wait()

    return bt_kernel


def kernel(x, strengths):
    info = plsc.get_sparse_core_info()
    fn = _make_kernel(info.num_cores, info.num_subcores)
    xi = x.astype(jnp.int32)
    return fn(xi[:, 0], xi[:, 1], strengths)


# speedup vs baseline: 1.5021x; 1.0033x over previous
import functools

import jax
import jax.numpy as jnp
from jax import lax
from jax.experimental import pallas as pl
from jax.experimental.pallas import tpu as pltpu
from jax.experimental.pallas import tpu_sc as plsc

BATCH = 16384
LANES = 16


def _make_kernel(num_cores, num_subcores):
    nw = num_cores * num_subcores
    pairs_per_w = BATCH // nw
    streams = pairs_per_w // 128
    groups = pairs_per_w // LANES

    mesh = plsc.VectorSubcoreMesh(core_axis_name="c", subcore_axis_name="s")

    @functools.partial(
        pl.kernel,
        mesh=mesh,
        out_type=jax.ShapeDtypeStruct((BATCH,), jnp.float32),
        scratch_types=[
            pltpu.VMEM((pairs_per_w,), jnp.int32),
            pltpu.VMEM((pairs_per_w,), jnp.int32),
            pltpu.VMEM((pairs_per_w,), jnp.float32),
            pltpu.VMEM((pairs_per_w,), jnp.float32),
            pltpu.VMEM((pairs_per_w,), jnp.float32),
            pltpu.SemaphoreType.DMA((2, 4)),
            pltpu.SemaphoreType.DMA((2, 4)),
            pltpu.SemaphoreType.DMA,
        ],
        compiler_params=pltpu.CompilerParams(
            needs_layout_passes=False,
            disable_bounds_checks=True,
            disable_semaphore_checks=True,
        ),
    )
    def bt_kernel(
        xw_hbm, xl_hbm, s_hbm, out_hbm,
        wi_v, li_v, wv, lv, out_v, ssem, gsem, osem,
    ):
        wid = lax.axis_index("s") * num_cores + lax.axis_index("c")
        base = wid * pairs_per_w
        sides = ((xw_hbm, wi_v, wv), (xl_hbm, li_v, lv))
        stage = [
            [
                pltpu.async_copy(
                    src.at[pl.ds(base + j * 128, 128)],
                    idx.at[pl.ds(j * 128, 128)],
                    ssem.at[side, j],
                )
                for j in range(streams)
            ]
            for side, (src, idx, _) in enumerate(sides)
        ]
        gather = [[None] * streams, [None] * streams]
        for j in range(streams):
            for side, (_, idx, dst) in enumerate(sides):
                stage[side][j].wait()
                gather[side][j] = pltpu.async_copy(
                    s_hbm.at[idx.at[pl.ds(j * 128, 128)]],
                    dst.at[pl.ds(j * 128, 128)],
                    gsem.at[side, j],
                )
        out_copies = []
        for j in range(streams):
            gather[0][j].wait()
            gather[1][j].wait()

            def chunk_body(i, _):
                sl = pl.ds(i * LANES, LANES)
                s_w = wv[sl]
                out_v[sl] = s_w / (s_w + lv[sl])
                return 0

            lax.fori_loop(8 * j, 8 * (j + 1), chunk_body, 0, unroll=2)
            out_copies.append(
                pltpu.async_copy(
                    out_v.at[pl.ds(j * 128, 128)],
                    out_hbm.at[pl.ds(base + j * 128, 128)],
                    osem,
                )
            )
        for c in out_copies:
            c.wait()

    return bt_kernel


def kernel(x, strengths):
    info = plsc.get_sparse_core_info()
    fn = _make_kernel(info.num_cores, info.num_subcores)
    xi = x.astype(jnp.int32)
    return fn(xi[:, 0], xi[:, 1], strengths)


# speedup vs baseline: 1.5056x; 1.0023x over previous
import functools

import jax
import jax.numpy as jnp
from jax import lax
from jax.experimental import pallas as pl
from jax.experimental.pallas import tpu as pltpu
from jax.experimental.pallas import tpu_sc as plsc

BATCH = 16384
LANES = 16


def _make_kernel(num_cores, num_subcores):
    nw = num_cores * num_subcores
    pairs_per_w = BATCH // nw
    streams = pairs_per_w // 128
    groups = pairs_per_w // LANES

    mesh = plsc.VectorSubcoreMesh(core_axis_name="c", subcore_axis_name="s")

    @functools.partial(
        pl.kernel,
        mesh=mesh,
        out_type=jax.ShapeDtypeStruct((BATCH,), jnp.float32),
        scratch_types=[
            pltpu.VMEM((pairs_per_w,), jnp.int32),
            pltpu.VMEM((pairs_per_w,), jnp.int32),
            pltpu.VMEM((pairs_per_w,), jnp.float32),
            pltpu.VMEM((pairs_per_w,), jnp.float32),
            pltpu.VMEM((pairs_per_w,), jnp.float32),
            pltpu.SemaphoreType.DMA((2, 4)),
            pltpu.SemaphoreType.DMA((2, 4)),
            pltpu.SemaphoreType.DMA,
        ],
        compiler_params=pltpu.CompilerParams(
            needs_layout_passes=False,
            disable_bounds_checks=True,
            disable_semaphore_checks=True,
        ),
    )
    def bt_kernel(
        xt_hbm, s_hbm, out_hbm,
        wi_v, li_v, wv, lv, out_v, ssem, gsem, osem,
    ):
        wid = lax.axis_index("s") * num_cores + lax.axis_index("c")
        base = wid * pairs_per_w
        sides = ((xt_hbm.at[0], wi_v, wv), (xt_hbm.at[1], li_v, lv))
        stage = [
            [
                pltpu.async_copy(
                    src.at[pl.ds(base + j * 128, 128)],
                    idx.at[pl.ds(j * 128, 128)],
                    ssem.at[side, j],
                )
                for j in range(streams)
            ]
            for side, (src, idx, _) in enumerate(sides)
        ]
        gather = [[None] * streams, [None] * streams]
        for j in range(streams):
            for side, (_, idx, dst) in enumerate(sides):
                stage[side][j].wait()
                gather[side][j] = pltpu.async_copy(
                    s_hbm.at[idx.at[pl.ds(j * 128, 128)]],
                    dst.at[pl.ds(j * 128, 128)],
                    gsem.at[side, j],
                )
        out_copies = []
        for j in range(streams):
            gather[0][j].wait()
            gather[1][j].wait()

            def chunk_body(i, _):
                sl = pl.ds(i * LANES, LANES)
                s_w = wv[sl]
                out_v[sl] = s_w / (s_w + lv[sl])
                return 0

            lax.fori_loop(8 * j, 8 * (j + 1), chunk_body, 0, unroll=2)
            out_copies.append(
                pltpu.async_copy(
                    out_v.at[pl.ds(j * 128, 128)],
                    out_hbm.at[pl.ds(base + j * 128, 128)],
                    osem,
                )
            )
        for c in out_copies:
            c.wait()

    return bt_kernel


def kernel(x, strengths):
    info = plsc.get_sparse_core_info()
    fn = _make_kernel(info.num_cores, info.num_subcores)
    return fn(x.astype(jnp.int32).T, strengths)
